# Initial kernel scaffold; baseline (speedup 1.0000x reference)
import functools

import jax
import jax.numpy as jnp
from jax import lax
from jax.experimental import pallas as pl
from jax.experimental.pallas import tpu as pltpu
from jax.experimental.pallas import tpu_sc as plsc

BB = 4
NP = 8192
DF = 256
GG = 1024
KK = 32
XP = 8

_NW = 32
_BIG = jnp.int32(1 << 30)


def _fps_body(xyz_ref, idx_ref, lc_ref):
    X = xyz_ref[:, 0]
    Y = xyz_ref[:, 1]
    Z = xyz_ref[:, 2]
    shp = (BB, 8, NP // 8)
    row = lax.broadcasted_iota(jnp.int32, shp, 1)
    col = lax.broadcasted_iota(jnp.int32, shp, 2)
    flat = row * (NP // 8) + col

    def step(t, carry):
        dists, far = carry
        sel = flat == far
        zero = jnp.float32(0.0)
        cx = jnp.sum(jnp.where(sel, X, zero), axis=(1, 2), keepdims=True)
        cy = jnp.sum(jnp.where(sel, Y, zero), axis=(1, 2), keepdims=True)
        cz = jnp.sum(jnp.where(sel, Z, zero), axis=(1, 2), keepdims=True)
        idx_ref[:, pl.ds(t, 1)] = far[:, 0, :]
        lc_ref[0, :, pl.ds(t, 1)] = cx[:, 0, :]
        lc_ref[1, :, pl.ds(t, 1)] = cy[:, 0, :]
        lc_ref[2, :, pl.ds(t, 1)] = cz[:, 0, :]
        dx = X - cx
        dy = Y - cy
        dz = Z - cz
        d = dx * dx + dy * dy + dz * dz
        dists = jnp.minimum(dists, d)
        m = jnp.max(dists, axis=(1, 2), keepdims=True)
        nxt = jnp.min(jnp.where(dists == m, flat, _BIG), axis=(1, 2),
                      keepdims=True)
        return dists, nxt

    init = (jnp.full(shp, 1e10, jnp.float32),
            jnp.zeros((BB, 1, 1), jnp.int32))
    lax.fori_loop(0, GG, step, init)


def _run_fps(xyzr4):
    return pl.pallas_call(
        _fps_body,
        out_shape=(jax.ShapeDtypeStruct((BB, GG), jnp.int32),
                   jax.ShapeDtypeStruct((3, BB, GG), jnp.float32)),
    )(xyzr4)


_BG = 256


def _knn_body(xyz_ref, lc_ref, out_ref):
    px = xyz_ref[0, 0:1, :]
    py = xyz_ref[0, 1:2, :]
    pz = xyz_ref[0, 2:3, :]
    lcx = lc_ref[0, :, 0:1]
    lcy = lc_ref[0, :, 1:2]
    lcz = lc_ref[0, :, 2:3]
    s2 = lcx * lcx + lcy * lcy + lcz * lcz
    d2 = px * px + py * py + pz * pz
    cross = lcx * px + lcy * py + lcz * pz
    sq = s2 + d2 - 2.0 * cross
    colid = lax.broadcasted_iota(jnp.int32, (_BG, NP), 1)

    def tk(j, d):
        m = jnp.min(d, axis=1, keepdims=True)
        idx = jnp.min(jnp.where(d == m, colid, _BIG), axis=1, keepdims=True)
        out_ref[0, :, pl.ds(j, 1)] = idx
        return jnp.where(colid == idx, jnp.float32(jnp.inf), d)

    lax.fori_loop(0, KK, tk, sq)


def _run_knn(xyzr, lcT):
    grid = (BB, GG // _BG)
    return pl.pallas_call(
        _knn_body,
        grid=grid,
        in_specs=[
            pl.BlockSpec((1, 3, NP), lambda b, j: (b, 0, 0)),
            pl.BlockSpec((1, _BG, 3), lambda b, j: (b, j, 0)),
        ],
        out_specs=pl.BlockSpec((1, _BG, KK), lambda b, j: (b, j, 0)),
        out_shape=jax.ShapeDtypeStruct((BB, GG, KK), jnp.int32),
    )(xyzr, lcT)


def _make_sc_gather(n_rows, d, ch):
    rows_w = n_rows // _NW
    n_ch = rows_w // ch
    assert rows_w % ch == 0 and n_rows % _NW == 0
    mesh = plsc.VectorSubcoreMesh(core_axis_name="c", subcore_axis_name="s")

    @functools.partial(
        pl.kernel, mesh=mesh,
        out_type=jax.ShapeDtypeStruct((n_rows, d), jnp.float32),
        scratch_types=[
            pltpu.VMEM((rows_w,), jnp.int32),
            pltpu.VMEM((ch, d), jnp.float32),
            pltpu.SemaphoreType.DMA,
        ],
    )
    def k(tbl_hbm, idx_hbm, out_hbm, idx_v, buf, sem):
        wid = lax.axis_index("s") * 2 + lax.axis_index("c")
        base = wid * rows_w
        pltpu.sync_copy(idx_hbm.at[pl.ds(base, rows_w)], idx_v)
        off = (wid // 8) * NP

        def addoff(i, _):
            sl = pl.ds(i * 16, 16)
            idx_v[sl] = idx_v[sl] + off
            return 0

        lax.fori_loop(0, rows_w // 16, addoff, 0)

        def chunk(c, _):
            s = c * ch
            pltpu.async_copy(tbl_hbm.at[idx_v.at[pl.ds(s, ch)]], buf,
                             sem).wait()
            pltpu.sync_copy(buf, out_hbm.at[pl.ds(base + s, ch)])
            return 0

        lax.fori_loop(0, n_ch, chunk, 0)

    return k


def _sums_body(raw_ref, ctr_ref, s1_ref, s2_ref):
    i = pl.program_id(0)
    rb = raw_ref.shape[0]
    d = raw_ref.shape[1]
    raw = raw_ref[...].reshape(rb // KK, KK, d)
    ctr = ctr_ref[...][:, None, :]
    diff = raw - ctr
    s1 = jnp.sum(diff)
    s2 = jnp.sum(diff * diff)

    @pl.when(i == 0)
    def _():
        s1_ref[0, 0] = jnp.float32(0.0)
        s2_ref[0, 0] = jnp.float32(0.0)

    s1_ref[0, 0] += s1
    s2_ref[0, 0] += s2


def _run_sums(raw, ctr, rb):
    n_rows, d = raw.shape
    grid = (n_rows // rb,)
    return pl.pallas_call(
        _sums_body,
        grid=grid,
        in_specs=[
            pl.BlockSpec((rb, d), lambda i: (i, 0)),
            pl.BlockSpec((rb // KK, d), lambda i: (i, 0)),
        ],
        out_specs=(pl.BlockSpec((1, 1), lambda i: (0, 0)),
                   pl.BlockSpec((1, 1), lambda i: (0, 0))),
        out_shape=(jax.ShapeDtypeStruct((1, 1), jnp.float32),
                   jax.ShapeDtypeStruct((1, 1), jnp.float32)),
    )(raw, ctr)


def _normx_body(raw_ref, ctr_ref, inv_ref, out_ref):
    rb = raw_ref.shape[0]
    inv = inv_ref[0]
    raw = raw_ref[...].reshape(rb // KK, KK, DF)
    ctr = ctr_ref[...]
    ctr3 = ctr[:, None, :]
    nrm = ((raw - ctr3) * inv).reshape(rb, DF)
    rep = jnp.broadcast_to(ctr3, (rb // KK, KK, DF)).reshape(rb, DF)
    out_ref[:, 0:DF] = nrm
    out_ref[:, DF:2 * DF] = rep


def _run_normx(raw, ctr, inv, rb):
    n_rows = raw.shape[0]
    return pl.pallas_call(
        _normx_body,
        grid=(n_rows // rb,),
        in_specs=[
            pl.BlockSpec((rb, DF), lambda i: (i, 0)),
            pl.BlockSpec((rb // KK, DF), lambda i: (i, 0)),
            pl.BlockSpec(memory_space=pltpu.SMEM),
        ],
        out_specs=pl.BlockSpec((rb, 2 * DF), lambda i: (i, 0)),
        out_shape=jax.ShapeDtypeStruct((n_rows, 2 * DF), jnp.float32),
    )(raw, ctr, inv)


def _normxyz_body(raw_ref, ctr_ref, inv_ref, out_ref):
    rb = raw_ref.shape[0]
    inv = inv_ref[0]
    raw = raw_ref[...].reshape(rb // KK, KK, XP)
    ctr3 = ctr_ref[...][:, None, :]
    nrm = ((raw - ctr3) * inv).reshape(rb, XP)
    out_ref[...] = nrm[:, 0:3]


def _run_normxyz(raw, ctr, inv, rb):
    n_rows = raw.shape[0]
    return pl.pallas_call(
        _normxyz_body,
        grid=(n_rows // rb,),
        in_specs=[
            pl.BlockSpec((rb, XP), lambda i: (i, 0)),
            pl.BlockSpec((rb // KK, XP), lambda i: (i, 0)),
            pl.BlockSpec(memory_space=pltpu.SMEM),
        ],
        out_specs=pl.BlockSpec((rb, 3), lambda i: (i, 0)),
        out_shape=jax.ShapeDtypeStruct((n_rows, 3), jnp.float32),
    )(raw, ctr, inv)


def _std_from_sums(s1, s2, m):
    mean = s1[0, 0] / m
    var = (s2[0, 0] - m * mean * mean) / (m - 1.0)
    return jnp.sqrt(var)


def kernel(xyz, x):
    xyzr = jnp.transpose(xyz, (0, 2, 1))
    xyzr4 = xyzr.reshape(BB, 3, 8, NP // 8)

    fps_i, lc3 = _run_fps(xyzr4)
    lc_xyz = jnp.transpose(lc3, (1, 2, 0))

    knn_idx = _run_knn(xyzr, lc_xyz)

    xall = x.reshape(BB * NP, DF)
    xyzp = jnp.pad(xyz, ((0, 0), (0, 0), (0, XP - 3))).reshape(BB * NP, XP)
    gidx = knn_idx.reshape(BB * GG * KK)
    fidx = fps_i.reshape(BB * GG)

    raw_x = _make_sc_gather(BB * GG * KK, DF, 128)(xall, gidx)
    lc_x = _make_sc_gather(BB * GG, DF, 128)(xall, fidx)
    raw_xyz = _make_sc_gather(BB * GG * KK, XP, 128)(xyzp, gidx)

    lc_xyzp = jnp.pad(lc_xyz, ((0, 0), (0, 0), (0, XP - 3))).reshape(
        BB * GG, XP)

    s1x, s2x = _run_sums(raw_x, lc_x, 2048)
    s1z, s2z = _run_sums(raw_xyz, lc_xyzp, 4096)

    m_x = float(BB * GG * KK * DF)
    m_z = float(BB * GG * KK * 3)
    inv_x = (1.0 / (_std_from_sums(s1x, s2x, m_x) + 1e-5)).reshape(1)
    inv_z = (1.0 / (_std_from_sums(s1z, s2z, m_z) + 1e-5)).reshape(1)

    knn_x = _run_normx(raw_x, lc_x, inv_x, 2048)
    knn_xyz = _run_normxyz(raw_xyz, lc_xyzp, inv_z, 4096)

    return (lc_xyz,
            lc_x.reshape(BB, GG, DF),
            knn_xyz.reshape(BB, GG, KK, 3),
            knn_x.reshape(BB, GG, KK, 2 * DF))


# speedup vs baseline: 4.9461x; 4.9461x over previous
import functools

import jax
import jax.numpy as jnp
from jax import lax
from jax.experimental import pallas as pl
from jax.experimental.pallas import tpu as pltpu
from jax.experimental.pallas import tpu_sc as plsc

BB = 4
NP = 8192
DF = 256
GG = 1024
KK = 32
XP = 128

_NW = 32
_BIG = 1 << 30


def _fps_body(xyz_ref, idx_ref, lc_ref):
    shp = (8, NP // 8)
    row = lax.broadcasted_iota(jnp.int32, shp, 0)
    col = lax.broadcasted_iota(jnp.int32, shp, 1)
    flat = row * (NP // 8) + col
    coords = [[xyz_ref[b, c] for c in range(3)] for b in range(BB)]

    def step(t, carry):
        dists, far = carry
        new_d, new_f = [], []
        for b in range(BB):
            Xb, Yb, Zb = coords[b]
            sel = flat == far[b]
            zero = jnp.float32(0.0)
            cx = jnp.sum(jnp.where(sel, Xb, zero), keepdims=True)
            cy = jnp.sum(jnp.where(sel, Yb, zero), keepdims=True)
            cz = jnp.sum(jnp.where(sel, Zb, zero), keepdims=True)
            idx_ref[pl.ds(t, 1), b:b + 1] = far[b]
            lc_ref[0, pl.ds(t, 1), b:b + 1] = cx
            lc_ref[1, pl.ds(t, 1), b:b + 1] = cy
            lc_ref[2, pl.ds(t, 1), b:b + 1] = cz
            dx = Xb - cx
            dy = Yb - cy
            dz = Zb - cz
            d = dx * dx + dy * dy + dz * dz
            db = jnp.minimum(dists[b], d)
            m = jnp.max(db, keepdims=True)
            nxt = jnp.min(jnp.where(db == m, flat, _BIG), keepdims=True)
            new_d.append(db)
            new_f.append(nxt)
        return tuple(new_d), tuple(new_f)

    init = (tuple(jnp.full(shp, 1e10, jnp.float32) for _ in range(BB)),
            tuple(jnp.zeros((1, 1), jnp.int32) for _ in range(BB)))
    lax.fori_loop(0, GG, step, init)


def _run_fps(xyzr4):
    return pl.pallas_call(
        _fps_body,
        out_shape=(jax.ShapeDtypeStruct((GG, BB), jnp.int32),
                   jax.ShapeDtypeStruct((3, GG, BB), jnp.float32)),
    )(xyzr4)


_BG = 128


def _bf16r(v):
    u = lax.bitcast_convert_type(v, jnp.uint32)
    r = (u + jnp.uint32(0x7FFF) + ((u >> 16) & jnp.uint32(1))) \
        & jnp.uint32(0xFFFF0000)
    return lax.bitcast_convert_type(r, jnp.float32)


def _knn_body(xyz_ref, lc_ref, out_ref):
    px = xyz_ref[0, :, 0:1]
    py = xyz_ref[0, :, 1:2]
    pz = xyz_ref[0, :, 2:3]
    lcx = lc_ref[0:1]
    lcy = lc_ref[1:2]
    lcz = lc_ref[2:3]
    s2 = lcx * lcx + lcy * lcy + lcz * lcz
    d2 = px * px + py * py + pz * pz
    cross = (_bf16r(lcx) * _bf16r(px) + _bf16r(lcy) * _bf16r(py)
             + _bf16r(lcz) * _bf16r(pz))
    sq = s2 + d2 - 2.0 * cross
    rowid = lax.broadcasted_iota(jnp.int32, (NP, _BG), 0)

    def tk(j, d):
        m = jnp.min(d, axis=0, keepdims=True)
        idx = jnp.min(jnp.where(d == m, rowid, _BIG), axis=0, keepdims=True)
        out_ref[0, pl.ds(j, 1), :] = idx
        return jnp.where(rowid == idx, jnp.float32(jnp.inf), d)

    lax.fori_loop(0, KK, tk, sq)


def _run_knn(xyz, lc3):
    grid = (BB, GG // _BG)
    return pl.pallas_call(
        _knn_body,
        grid=grid,
        in_specs=[
            pl.BlockSpec((1, NP, 3), lambda b, j: (b, 0, 0)),
            pl.BlockSpec((3, _BG), lambda b, j: (0, b * (GG // _BG) + j)),
        ],
        out_specs=pl.BlockSpec((1, KK, _BG), lambda b, j: (b, 0, j)),
        out_shape=jax.ShapeDtypeStruct((BB, KK, GG), jnp.int32),
    )(xyz, lc3)


def _make_sc_gather(n_rows, d, ch):
    rows_w = n_rows // _NW
    n_ch = rows_w // ch
    assert rows_w % ch == 0 and n_rows % _NW == 0
    mesh = plsc.VectorSubcoreMesh(core_axis_name="c", subcore_axis_name="s")

    @functools.partial(
        pl.kernel, mesh=mesh,
        out_type=jax.ShapeDtypeStruct((n_rows, d), jnp.float32),
        scratch_types=[
            pltpu.VMEM((rows_w,), jnp.int32),
            pltpu.VMEM((ch, d), jnp.float32),
            pltpu.SemaphoreType.DMA,
        ],
    )
    def k(tbl_hbm, idx_hbm, out_hbm, idx_v, buf, sem):
        wid = lax.axis_index("s") * 2 + lax.axis_index("c")
        base = wid * rows_w
        pltpu.sync_copy(idx_hbm.at[pl.ds(base, rows_w)], idx_v)
        off = (wid // 8) * NP

        def addoff(i, _):
            sl = pl.ds(i * 16, 16)
            idx_v[sl] = idx_v[sl] + off
            return 0

        lax.fori_loop(0, rows_w // 16, addoff, 0)

        def chunk(c, _):
            s = c * ch
            pltpu.async_copy(tbl_hbm.at[idx_v.at[pl.ds(s, ch)]], buf,
                             sem).wait()
            pltpu.sync_copy(buf, out_hbm.at[pl.ds(base + s, ch)])
            return 0

        lax.fori_loop(0, n_ch, chunk, 0)

    return k


def _sums_body(raw_ref, ctr_ref, s1_ref, s2_ref):
    i = pl.program_id(0)
    rb = raw_ref.shape[0]
    d = raw_ref.shape[1]
    raw = raw_ref[...].reshape(rb // KK, KK, d)
    ctr = ctr_ref[...][:, None, :]
    diff = raw - ctr
    s1 = jnp.sum(diff)
    s2 = jnp.sum(diff * diff)

    @pl.when(i == 0)
    def _():
        s1_ref[0, 0] = jnp.float32(0.0)
        s2_ref[0, 0] = jnp.float32(0.0)

    s1_ref[0, 0] += s1
    s2_ref[0, 0] += s2


def _run_sums(raw, ctr, rb):
    n_rows, d = raw.shape
    grid = (n_rows // rb,)
    return pl.pallas_call(
        _sums_body,
        grid=grid,
        in_specs=[
            pl.BlockSpec((rb, d), lambda i: (i, 0)),
            pl.BlockSpec((rb // KK, d), lambda i: (i, 0)),
        ],
        out_specs=(pl.BlockSpec(memory_space=pltpu.SMEM),
                   pl.BlockSpec(memory_space=pltpu.SMEM)),
        out_shape=(jax.ShapeDtypeStruct((1, 1), jnp.float32),
                   jax.ShapeDtypeStruct((1, 1), jnp.float32)),
    )(raw, ctr)


def _normx_body(raw_ref, ctr_ref, inv_ref, out_ref):
    rb = raw_ref.shape[0]
    inv = inv_ref[0]
    raw = raw_ref[...].reshape(rb // KK, KK, DF)
    ctr = ctr_ref[...]
    ctr3 = ctr[:, None, :]
    nrm = ((raw - ctr3) * inv).reshape(rb, DF)
    rep = jnp.broadcast_to(ctr3, (rb // KK, KK, DF)).reshape(rb, DF)
    out_ref[:, 0:DF] = nrm
    out_ref[:, DF:2 * DF] = rep


def _run_normx(raw, ctr, inv, rb):
    n_rows = raw.shape[0]
    return pl.pallas_call(
        _normx_body,
        grid=(n_rows // rb,),
        in_specs=[
            pl.BlockSpec((rb, DF), lambda i: (i, 0)),
            pl.BlockSpec((rb // KK, DF), lambda i: (i, 0)),
            pl.BlockSpec(memory_space=pltpu.SMEM),
        ],
        out_specs=pl.BlockSpec((rb, 2 * DF), lambda i: (i, 0)),
        out_shape=jax.ShapeDtypeStruct((n_rows, 2 * DF), jnp.float32),
    )(raw, ctr, inv)


def _normxyz_body(raw_ref, ctr_ref, inv_ref, out_ref):
    rb = raw_ref.shape[0]
    inv = inv_ref[0]
    raw = raw_ref[...].reshape(rb // KK, KK, XP)
    ctr3 = ctr_ref[...][:, None, :]
    nrm = ((raw - ctr3) * inv).reshape(rb, XP)
    out_ref[...] = nrm[:, 0:3]


def _run_normxyz(raw, ctr, inv, rb):
    n_rows = raw.shape[0]
    return pl.pallas_call(
        _normxyz_body,
        grid=(n_rows // rb,),
        in_specs=[
            pl.BlockSpec((rb, XP), lambda i: (i, 0)),
            pl.BlockSpec((rb // KK, XP), lambda i: (i, 0)),
            pl.BlockSpec(memory_space=pltpu.SMEM),
        ],
        out_specs=pl.BlockSpec((rb, 3), lambda i: (i, 0)),
        out_shape=jax.ShapeDtypeStruct((n_rows, 3), jnp.float32),
    )(raw, ctr, inv)


def _std_from_sums(s1, s2, m):
    mean = s1[0, 0] / m
    var = (s2[0, 0] - m * mean * mean) / (m - 1.0)
    return jnp.sqrt(var)


def kernel(xyz, x):
    xyzr = jnp.transpose(xyz, (0, 2, 1))
    xyzr4 = xyzr.reshape(BB, 3, 8, NP // 8)

    fps_gb, lc3 = _run_fps(xyzr4)
    fps_i = jnp.transpose(fps_gb, (1, 0))
    lc_xyz = jnp.transpose(lc3, (2, 1, 0))
    lc3bg = jnp.transpose(lc3, (0, 2, 1)).reshape(3, BB * GG)

    knn_bkg = _run_knn(xyz, lc3bg)
    knn_idx = jnp.transpose(knn_bkg, (0, 2, 1))

    xall = x.reshape(BB * NP, DF)
    xyzp = jnp.pad(xyz, ((0, 0), (0, 0), (0, XP - 3))).reshape(BB * NP, XP)
    gidx = knn_idx.reshape(BB * GG * KK)
    fidx = fps_i.reshape(BB * GG)

    raw_x = _make_sc_gather(BB * GG * KK, DF, 128)(xall, gidx)
    lc_x = _make_sc_gather(BB * GG, DF, 128)(xall, fidx)
    raw_xyz = _make_sc_gather(BB * GG * KK, XP, 128)(xyzp, gidx)

    lc_xyzp = jnp.pad(lc_xyz, ((0, 0), (0, 0), (0, XP - 3))).reshape(
        BB * GG, XP)

    s1x, s2x = _run_sums(raw_x, lc_x, 2048)
    s1z, s2z = _run_sums(raw_xyz, lc_xyzp, 4096)

    m_x = float(BB * GG * KK * DF)
    m_z = float(BB * GG * KK * 3)
    inv_x = (1.0 / (_std_from_sums(s1x, s2x, m_x) + 1e-5)).reshape(1)
    inv_z = (1.0 / (_std_from_sums(s1z, s2z, m_z) + 1e-5)).reshape(1)

    knn_x = _run_normx(raw_x, lc_x, inv_x, 2048)
    knn_xyz = _run_normxyz(raw_xyz, lc_xyzp, inv_z, 4096)

    return (lc_xyz,
            lc_x.reshape(BB, GG, DF),
            knn_xyz.reshape(BB, GG, KK, 3),
            knn_x.reshape(BB, GG, KK, 2 * DF))
